# baseline (speedup 1.0000x reference)
import jax
import jax.numpy as jnp
from jax import lax
from jax.experimental import pallas as pl
from jax.experimental.pallas import tpu as pltpu
from jax.experimental.pallas import tpu_sc as plsc

N_ROWS = 65536
D = 512
SIZE = int(0.01 * N_ROWS)

NUM_WORKERS = 32
ROWS_PER_WORKER = -(-SIZE // (NUM_WORKERS * 8)) * 8
PAD_SIZE = NUM_WORKERS * ROWS_PER_WORKER
LANES = 16

HEAD = PAD_SIZE
BLOCK = 6144


def _build_perms():
    pkey = jax.random.key(1234)
    keys = jax.random.split(pkey, SIZE)
    perms = jax.vmap(lambda k: jax.random.permutation(k, D))(keys)
    perms = perms.astype(jnp.int32)
    pad = jnp.tile(jnp.arange(D, dtype=jnp.int32), (PAD_SIZE - SIZE, 1))
    return jnp.concatenate([perms, pad], axis=0)


def _sc_body(x_hbm, perm_hbm, out_hbm, vx, vp, vo):
    wid = lax.axis_index("s") * 2 + lax.axis_index("c")
    start = wid * ROWS_PER_WORKER
    pltpu.sync_copy(x_hbm.at[pl.ds(start, ROWS_PER_WORKER)], vx)
    pltpu.sync_copy(perm_hbm.at[pl.ds(start, ROWS_PER_WORKER)], vp)

    for r in range(ROWS_PER_WORKER):
        row_view = vx.at[r]
        for j in range(D // LANES):
            cols = vp[r, pl.ds(j * LANES, LANES)]
            vo[r, pl.ds(j * LANES, LANES)] = plsc.load_gather(row_view, [cols])

    pltpu.sync_copy(vo, out_hbm.at[pl.ds(start, ROWS_PER_WORKER)])


_sc_gather = pl.kernel(
    _sc_body,
    out_type=jax.ShapeDtypeStruct((PAD_SIZE, D), jnp.float32),
    mesh=plsc.VectorSubcoreMesh(core_axis_name="c", subcore_axis_name="s"),
    compiler_params=pltpu.CompilerParams(
        use_tc_tiling_on_sc=False, needs_layout_passes=False
    ),
    scratch_types=[
        pltpu.VMEM((ROWS_PER_WORKER, D), jnp.float32),
        pltpu.VMEM((ROWS_PER_WORKER, D), jnp.int32),
        pltpu.VMEM((ROWS_PER_WORKER, D), jnp.float32),
    ],
)


def _tc_body(x_ref, out_ref):
    out_ref[...] = x_ref[...]


def kernel(x):
    y = pl.pallas_call(
        _tc_body,
        grid=(-(-N_ROWS // BLOCK),),
        in_specs=[pl.BlockSpec((BLOCK, D), lambda i: (i, 0))],
        out_specs=pl.BlockSpec((BLOCK, D), lambda i: (i, 0)),
        out_shape=jax.ShapeDtypeStruct((N_ROWS, D), jnp.float32),
        compiler_params=pltpu.CompilerParams(vmem_limit_bytes=100 * 1024 * 1024),
    )(x)
    s = _sc_gather(x[:PAD_SIZE], _build_perms())
    return lax.dynamic_update_slice(y, s, (0, 0))


# speedup vs baseline: 1.8023x; 1.8023x over previous
import jax
import jax.numpy as jnp
import numpy as np
from jax import lax
from jax.experimental import pallas as pl
from jax.experimental.pallas import tpu as pltpu
from jax.experimental.pallas import tpu_sc as plsc

N_ROWS = 65536
D = 512
SIZE = int(0.01 * N_ROWS)

NUM_WORKERS = 32
ROWS_PER_WORKER = -(-SIZE // (NUM_WORKERS * 8)) * 8
PAD_SIZE = NUM_WORKERS * ROWS_PER_WORKER
LANES = 16

HEAD = PAD_SIZE
BLOCK = 6144


def _threefry2x32(k1, k2, x0, x1):
    k1 = np.uint32(k1)
    k2 = np.uint32(k2)
    x0 = x0.astype(np.uint32)
    x1 = x1.astype(np.uint32)
    rot0 = (13, 15, 26, 6)
    rot1 = (17, 29, 16, 24)
    ks0, ks1 = k1, k2
    ks2 = np.uint32(k1 ^ k2 ^ np.uint32(0x1BD11BDA))

    def _rl(v, r):
        return ((v << np.uint32(r)) | (v >> np.uint32(32 - r))).astype(np.uint32)

    def _rounds(x0, x1, rots):
        for r in rots:
            x0 = (x0 + x1).astype(np.uint32)
            x1 = x0 ^ _rl(x1, r)
        return x0, x1

    x0 = (x0 + ks0).astype(np.uint32)
    x1 = (x1 + ks1).astype(np.uint32)
    for i, (ka, kb, rots) in enumerate(
        [(ks1, ks2, rot0), (ks2, ks0, rot1), (ks0, ks1, rot0),
         (ks1, ks2, rot1), (ks2, ks0, rot0)]
    ):
        x0, x1 = _rounds(x0, x1, rots)
        x0 = (x0 + ka).astype(np.uint32)
        x1 = (x1 + kb + np.uint32(i + 1)).astype(np.uint32)
    return x0, x1


def _build_perms():
    b1, b2 = _threefry2x32(
        0, 1234, np.zeros(SIZE, np.uint32), np.arange(SIZE, dtype=np.uint32)
    )
    perms = np.empty((PAD_SIZE, D), np.int32)
    two = np.arange(2, dtype=np.uint32)
    cnt = np.arange(D, dtype=np.uint32)
    zeros2 = np.zeros(2, np.uint32)
    zerosd = np.zeros(D, np.uint32)
    for i in range(SIZE):
        s1, s2 = _threefry2x32(b1[i], b2[i], zeros2, two)
        r1, r2 = _threefry2x32(s1[1], s2[1], zerosd, cnt)
        perms[i] = np.argsort(r1 ^ r2, kind="stable").astype(np.int32)
    perms[SIZE:] = np.arange(D, dtype=np.int32)
    return perms


_PERMS = _build_perms()


def _sc_body(x_hbm, perm_hbm, out_hbm, vx, vp, vo):
    wid = lax.axis_index("s") * 2 + lax.axis_index("c")
    start = wid * ROWS_PER_WORKER
    pltpu.sync_copy(x_hbm.at[pl.ds(start, ROWS_PER_WORKER)], vx)
    pltpu.sync_copy(perm_hbm.at[pl.ds(start, ROWS_PER_WORKER)], vp)

    for r in range(ROWS_PER_WORKER):
        row_view = vx.at[r]
        for j in range(D // LANES):
            cols = vp[r, pl.ds(j * LANES, LANES)]
            vo[r, pl.ds(j * LANES, LANES)] = plsc.load_gather(row_view, [cols])

    pltpu.sync_copy(vo, out_hbm.at[pl.ds(start, ROWS_PER_WORKER)])


_sc_gather = pl.kernel(
    _sc_body,
    out_type=jax.ShapeDtypeStruct((PAD_SIZE, D), jnp.float32),
    mesh=plsc.VectorSubcoreMesh(core_axis_name="c", subcore_axis_name="s"),
    compiler_params=pltpu.CompilerParams(
        use_tc_tiling_on_sc=False, needs_layout_passes=False
    ),
    scratch_types=[
        pltpu.VMEM((ROWS_PER_WORKER, D), jnp.float32),
        pltpu.VMEM((ROWS_PER_WORKER, D), jnp.int32),
        pltpu.VMEM((ROWS_PER_WORKER, D), jnp.float32),
    ],
)


def _tc_body(x_ref, out_ref):
    out_ref[...] = x_ref[...]


def kernel(x):
    y = pl.pallas_call(
        _tc_body,
        grid=(-(-N_ROWS // BLOCK),),
        in_specs=[pl.BlockSpec((BLOCK, D), lambda i: (i, 0))],
        out_specs=pl.BlockSpec((BLOCK, D), lambda i: (i, 0)),
        out_shape=jax.ShapeDtypeStruct((N_ROWS, D), jnp.float32),
        compiler_params=pltpu.CompilerParams(vmem_limit_bytes=100 * 1024 * 1024),
    )(x)
    s = _sc_gather(x[:PAD_SIZE], jnp.asarray(_PERMS))
    return lax.dynamic_update_slice(y, s, (0, 0))
